# baseline (speedup 1.0000x reference)
import functools

import jax
import jax.numpy as jnp
from jax import lax
from jax.experimental import pallas as pl
from jax.experimental.pallas import tpu as pltpu
from jax.experimental.pallas import tpu_sc as plsc

N = 10000
E = 160000
D = 256
DE = 16
CUTOFF = 5.0

NC, NS, L = 2, 16, 16
NW = NC * NS
EPW = E // NW
BK = 40

_sc_mesh = plsc.VectorSubcoreMesh(core_axis_name="c", subcore_axis_name="s")


@functools.partial(
    pl.kernel,
    out_type=jax.ShapeDtypeStruct((E, D), jnp.float32),
    mesh=_sc_mesh,
    scratch_types=[
        pltpu.VMEM((EPW,), jnp.int32),
        pltpu.VMEM((BK, D), jnp.float32),
        pltpu.SemaphoreType.DMA,
    ],
)
def _sc_gather_s(s_hbm, col_hbm, out_hbm, idx_v, rows_v, sem):
    wid = lax.axis_index("s") * NC + lax.axis_index("c")
    base = wid * EPW
    pltpu.sync_copy(col_hbm.at[pl.ds(base, EPW)], idx_v)

    def body(b, carry):
        pltpu.async_copy(s_hbm.at[idx_v.at[pl.ds(b * BK, BK)]], rows_v, sem).wait()
        pltpu.sync_copy(rows_v, out_hbm.at[pl.ds(base + b * BK, BK)])
        return carry

    lax.fori_loop(0, EPW // BK, body, 0)


def _silu(x):
    return x * jax.nn.sigmoid(x)


BE = 256


def _tc_dense_body(s_col_ref, attr_ref, dist_ref, diff_ref,
                   W1s_ref, b1s_ref, W2s_ref, b2s_ref,
                   W1f_ref, b1f_ref, W2f_ref, b2f_ref,
                   gates_ref, msgs_ref, dir8_ref):
    dist = dist_ref[...]
    attr = attr_ref[...]
    freqs = jax.lax.broadcasted_iota(jnp.int32, (1, 20), 1).astype(jnp.float32) + 1.0
    radial = jnp.sin(dist * freqs * (jnp.pi / CUTOFF))
    cutoff_w = 0.5 * (jnp.cos(dist * (jnp.pi / CUTOFF)) + 1.0)

    h_f = _silu(jnp.dot(radial, W1f_ref[0:20, :], preferred_element_type=jnp.float32)
                + jnp.dot(attr, W1f_ref[20:36, :], preferred_element_type=jnp.float32)
                + b1f_ref[...])
    fw = (jnp.dot(h_f, W2f_ref[...], preferred_element_type=jnp.float32)
          + b2f_ref[...]) * cutoff_w

    h_s = _silu(jnp.dot(s_col_ref[...], W1s_ref[0:D, :], preferred_element_type=jnp.float32)
                + jnp.dot(attr, W1s_ref[D:D + DE, :], preferred_element_type=jnp.float32)
                + b1s_ref[...])
    s_j = jnp.dot(h_s, W2s_ref[...], preferred_element_type=jnp.float32) + b2s_ref[...]

    prod = s_j * fw
    gates_ref[...] = prod[:, 0:2 * D]
    msgs_ref[...] = prod[:, 2 * D:3 * D]
    dir3 = diff_ref[...] / (dist + 1e-07)
    dir8_ref[...] = jnp.concatenate(
        [dir3, jnp.zeros((BE, 5), jnp.float32)], axis=1)


def _tc_dense(s_col, edge_attr, edge_dist, edge_diff,
              W1s, b1s, W2s, b2s, W1f, b1f, W2f, b2f):
    nblk = E // BE
    full = lambda shape: pl.BlockSpec(shape, lambda i: (0, 0))
    grid_spec = pl.GridSpec(
        grid=(nblk,),
        in_specs=[
            pl.BlockSpec((BE, D), lambda i: (i, 0)),
            pl.BlockSpec((BE, DE), lambda i: (i, 0)),
            pl.BlockSpec((BE, 1), lambda i: (i, 0)),
            pl.BlockSpec((BE, 3), lambda i: (i, 0)),
            full((D + DE, D)), full((1, D)), full((D, 3 * D)), full((1, 3 * D)),
            full((36, D)), full((1, D)), full((D, 3 * D)), full((1, 3 * D)),
        ],
        out_specs=[
            pl.BlockSpec((BE, 2 * D), lambda i: (i, 0)),
            pl.BlockSpec((BE, D), lambda i: (i, 0)),
            pl.BlockSpec((BE, 8), lambda i: (i, 0)),
        ],
    )
    return pl.pallas_call(
        _tc_dense_body,
        grid_spec=grid_spec,
        out_shape=[
            jax.ShapeDtypeStruct((E, 2 * D), jnp.float32),
            jax.ShapeDtypeStruct((E, D), jnp.float32),
            jax.ShapeDtypeStruct((E, 8), jnp.float32),
        ],
    )(s_col, edge_attr, edge_dist.reshape(E, 1), edge_diff,
      W1s, b1s.reshape(1, D), W2s, b2s.reshape(1, 3 * D),
      W1f, b1f.reshape(1, D), W2f, b2f.reshape(1, 3 * D))


def kernel(s, vec, edge_index, edge_diff, edge_dist, edge_attr,
           W1s, b1s, W2s, b2s, W1f, b1f, W2f, b2f):
    row = edge_index[0]
    col = edge_index[1]

    s_col = _sc_gather_s(s, col)
    gates, msg_s, dir8 = _tc_dense(s_col, edge_attr, edge_dist, edge_diff,
                                   W1s, b1s, W2s, b2s, W1f, b1f, W2f, b2f)
    gate_vec = gates[:, :D]
    gate_edge = gates[:, D:]
    edge_dir = dir8[:, :3]
    vec_col = jnp.take(vec, col, axis=0)
    msg_vec = vec_col * gate_vec[:, None, :] + edge_dir[:, :, None] * gate_edge[:, None, :]
    s_out = jax.ops.segment_sum(msg_s, row, num_segments=N)
    vec_out = jax.ops.segment_sum(msg_vec, row, num_segments=N)
    return (s + s_out, vec + vec_out)


# speedup vs baseline: 4.5273x; 4.5273x over previous
import functools

import jax
import jax.numpy as jnp
from jax import lax
from jax.experimental import pallas as pl
from jax.experimental.pallas import tpu as pltpu
from jax.experimental.pallas import tpu_sc as plsc

N = 10000
E = 160000
D = 256
DE = 16
CUTOFF = 5.0

NC, NS, L = 2, 16, 16
NW = NC * NS
EPW = E // NW
BK = 40

_sc_mesh = plsc.VectorSubcoreMesh(core_axis_name="c", subcore_axis_name="s")
_sc_params = pltpu.CompilerParams(needs_layout_passes=False)


@functools.partial(
    pl.kernel,
    out_type=jax.ShapeDtypeStruct((E, D), jnp.float32),
    mesh=_sc_mesh,
    scratch_types=[
        pltpu.VMEM((EPW,), jnp.int32),
        pltpu.VMEM((BK, D), jnp.float32),
        pltpu.SemaphoreType.DMA,
    ],
)
def _sc_gather_s(s_hbm, col_hbm, out_hbm, idx_v, rows_v, sem):
    wid = lax.axis_index("s") * NC + lax.axis_index("c")
    base = wid * EPW
    pltpu.sync_copy(col_hbm.at[pl.ds(base, EPW)], idx_v)

    def body(b, carry):
        pltpu.async_copy(s_hbm.at[idx_v.at[pl.ds(b * BK, BK)]], rows_v, sem).wait()
        pltpu.sync_copy(rows_v, out_hbm.at[pl.ds(base + b * BK, BK)])
        return carry

    lax.fori_loop(0, EPW // BK, body, 0)


NPT_S = 320
SBK = 64
SCH = 2000
NCH = E // SCH
IDCAP = SCH + 2 * SBK


def _vgather(v, idx):
    dnums = lax.GatherDimensionNumbers(
        offset_dims=(), collapsed_slice_dims=(0,), start_index_map=(0,))
    return lax.gather(v, idx[:, None], dnums, (1,),
                      mode=lax.GatherScatterMode.PROMISE_IN_BOUNDS)


def _compact_group(rowc, idb, tgb, kb, g, cn, lo, hi, trash, one16, zero16, lanes,
                   extra_src=None, extra_dst=None):
    r = rowc[pl.ds(g * L, L)]
    m = (r >= lo) & (r < hi)
    m32 = jnp.where(m, one16, zero16)
    pos = cn + plsc.cumsum(m32) - 1
    ids = lanes + (kb + g * L)
    plsc.store_scatter(idb, [pos], ids, mask=m)
    t = jnp.minimum(jnp.maximum(r - lo, 0), trash)
    plsc.store_scatter(tgb, [pos], t, mask=m)
    if extra_src is not None:
        plsc.store_scatter(extra_dst, [pos], extra_src[pl.ds(g * L, L)], mask=m)
    return cn + jnp.sum(m32)


@functools.partial(
    pl.kernel,
    out_type=jax.ShapeDtypeStruct((N, D), jnp.float32),
    mesh=_sc_mesh,
    compiler_params=_sc_params,
    scratch_types=[
        pltpu.VMEM((NPT_S + 8, D), jnp.float32),
        pltpu.VMEM((SCH,), jnp.int32),
        pltpu.VMEM((IDCAP,), jnp.int32),
        pltpu.VMEM((IDCAP,), jnp.int32),
        pltpu.VMEM((SBK, D), jnp.float32),
        pltpu.SemaphoreType.DMA,
    ],
)
def _sc_aggregate_s(msg_hbm, row_hbm, s_hbm, out_hbm, acc, rowc, idb, tgb, mbuf, sem):
    w = lax.axis_index("s") * NC + lax.axis_index("c")
    lo = w * NPT_S
    nrows = jnp.minimum(N - lo, NPT_S)

    zero16 = jnp.full((L,), 0, jnp.int32)
    one16 = jnp.full((L,), 1, jnp.int32)
    lanes = lax.broadcasted_iota(jnp.int32, (L,), 0)

    def initloop(i, _):
        pltpu.sync_copy(s_hbm.at[pl.ds(lo + i * 80, 80)], acc.at[pl.ds(i * 80, 80)])
        return _
    lax.fori_loop(0, nrows // 80, initloop, 0)

    def process_batches(cnt):
        nb = cnt // SBK

        def batch(b, _):
            pltpu.async_copy(msg_hbm.at[idb.at[pl.ds(b * SBK, SBK)]], mbuf, sem).wait()

            def group16(g, _2):
                tgt16 = tgb[pl.ds(b * SBK + g * L, L)]

                def edge(e16, _3):
                    rsplat = _vgather(tgt16, one16 * e16)
                    e = g * L + e16
                    for j in range(D // L):
                        v = mbuf[e, pl.ds(j * L, L)]
                        plsc.addupdate_scatter(acc, [rsplat, lanes + j * L], v)
                    return _3
                lax.fori_loop(0, L, edge, 0)
                return _2
            lax.fori_loop(0, SBK // L, group16, 0)
            return _
        lax.fori_loop(0, nb, batch, 0)

        def shift(g, _):
            idb[pl.ds(g * L, L)] = idb[pl.ds(nb * SBK + g * L, L)]
            tgb[pl.ds(g * L, L)] = tgb[pl.ds(nb * SBK + g * L, L)]
            return _
        lax.fori_loop(0, SBK // L, shift, 0)
        return cnt - nb * SBK

    def chunk(k, cnt):
        kb = lax.rem(k + 2 * w, NCH) * SCH
        pltpu.sync_copy(row_hbm.at[pl.ds(kb, SCH)], rowc)

        def group(g, cn):
            return _compact_group(rowc, idb, tgb, kb, g, cn, lo, lo + nrows,
                                  NPT_S, one16, zero16, lanes)
        cnt = lax.fori_loop(0, SCH // L, group, cnt)
        return process_batches(cnt)

    cnt = lax.fori_loop(0, NCH, chunk, jnp.int32(0))

    def pad(g, _):
        pos = cnt + lanes + g * L
        plsc.store_scatter(idb, [pos], zero16)
        plsc.store_scatter(tgb, [pos], one16 * NPT_S)
        return _
    lax.fori_loop(0, SBK // L, pad, 0)
    process_batches(((cnt + SBK - 1) // SBK) * SBK)

    def wb(i, _):
        pltpu.sync_copy(acc.at[pl.ds(i * 80, 80)], out_hbm.at[pl.ds(lo + i * 80, 80)])
        return _
    lax.fori_loop(0, nrows // 80, wb, 0)


NPT_V = 80
NPASS = 4
VBK = 32
VIDCAP = SCH + 2 * VBK
DV = 3 * D


@functools.partial(
    pl.kernel,
    out_type=jax.ShapeDtypeStruct((N, DV), jnp.float32),
    mesh=_sc_mesh,
    compiler_params=_sc_params,
    scratch_types=[
        pltpu.VMEM((NPT_V + 8, DV), jnp.float32),
        pltpu.VMEM((SCH,), jnp.int32),
        pltpu.VMEM((SCH,), jnp.int32),
        pltpu.VMEM((VIDCAP,), jnp.int32),
        pltpu.VMEM((VIDCAP,), jnp.int32),
        pltpu.VMEM((VIDCAP,), jnp.int32),
        pltpu.VMEM((VBK, 2 * D), jnp.float32),
        pltpu.VMEM((VBK, DV), jnp.float32),
        pltpu.VMEM((VBK, 128), jnp.float32),
        pltpu.SemaphoreType.DMA,
    ],
)
def _sc_aggregate_v(gates_hbm, dir8_hbm, row_hbm, col_hbm, vec_hbm, out_hbm,
                    acc, rowc, colc, idb, tgb, clb, gbuf, vbuf, dbuf, sem):
    w = lax.axis_index("s") * NC + lax.axis_index("c")

    zero16 = jnp.full((L,), 0, jnp.int32)
    one16 = jnp.full((L,), 1, jnp.int32)
    lanes = lax.broadcasted_iota(jnp.int32, (L,), 0)

    def one_pass(p, _):
        lo = p * (NW * NPT_V) + w * NPT_V
        nrows = jnp.maximum(jnp.minimum(N - lo, NPT_V), 0)

        def initloop(i, _i):
            pltpu.sync_copy(vec_hbm.at[pl.ds(lo + i * 8, 8)], acc.at[pl.ds(i * 8, 8)])
            return _i
        lax.fori_loop(0, nrows // 8, initloop, 0)

        def process_batches(cnt):
            nb = cnt // VBK

            def batch(b, _b):
                pltpu.async_copy(gates_hbm.at[idb.at[pl.ds(b * VBK, VBK)]], gbuf, sem).wait()
                pltpu.async_copy(dir8_hbm.at[idb.at[pl.ds(b * VBK, VBK)]], dbuf, sem).wait()
                pltpu.async_copy(vec_hbm.at[clb.at[pl.ds(b * VBK, VBK)]], vbuf, sem).wait()

                def group16(g, _2):
                    tgt16 = tgb[pl.ds(b * VBK + g * L, L)]

                    def edge(e16, _3):
                        rsplat = _vgather(tgt16, one16 * e16)
                        e = g * L + e16
                        dvec = dbuf[e, pl.ds(0, L)]
                        for k in range(3):
                            dsplat = _vgather(dvec, zero16 + k)
                            for j in range(D // L):
                                off = k * D + j * L
                                v = vbuf[e, pl.ds(off, L)]
                                gv = gbuf[e, pl.ds(j * L, L)]
                                ge = gbuf[e, pl.ds(D + j * L, L)]
                                val = v * gv + dsplat * ge
                                plsc.addupdate_scatter(acc, [rsplat, lanes + off], val)
                        return _3
                    lax.fori_loop(0, L, edge, 0)
                    return _2
                lax.fori_loop(0, VBK // L, group16, 0)
                return _b
            lax.fori_loop(0, nb, batch, 0)

            def shift(g, _s):
                idb[pl.ds(g * L, L)] = idb[pl.ds(nb * VBK + g * L, L)]
                tgb[pl.ds(g * L, L)] = tgb[pl.ds(nb * VBK + g * L, L)]
                clb[pl.ds(g * L, L)] = clb[pl.ds(nb * VBK + g * L, L)]
                return _s
            lax.fori_loop(0, VBK // L, shift, 0)
            return cnt - nb * VBK

        def chunk(k, cnt):
            kb = lax.rem(k + 2 * w + 7 * p, NCH) * SCH
            pltpu.sync_copy(row_hbm.at[pl.ds(kb, SCH)], rowc)
            pltpu.sync_copy(col_hbm.at[pl.ds(kb, SCH)], colc)

            def group(g, cn):
                return _compact_group(rowc, idb, tgb, kb, g, cn, lo, lo + nrows,
                                      NPT_V, one16, zero16, lanes,
                                      extra_src=colc, extra_dst=clb)
            cnt = lax.fori_loop(0, SCH // L, group, cnt)
            return process_batches(cnt)

        cnt = lax.fori_loop(0, NCH, chunk, jnp.int32(0))

        def pad(g, _p):
            pos = cnt + lanes + g * L
            plsc.store_scatter(idb, [pos], zero16)
            plsc.store_scatter(tgb, [pos], one16 * NPT_V)
            plsc.store_scatter(clb, [pos], zero16)
            return _p
        lax.fori_loop(0, VBK // L, pad, 0)
        process_batches(((cnt + VBK - 1) // VBK) * VBK)

        def wb(i, _w):
            pltpu.sync_copy(acc.at[pl.ds(i * 8, 8)], out_hbm.at[pl.ds(lo + i * 8, 8)])
            return _w
        lax.fori_loop(0, nrows // 8, wb, 0)
        return _

    lax.fori_loop(0, NPASS, one_pass, 0)


def _silu(x):
    return x * jax.nn.sigmoid(x)


BE = 256


def _tc_dense_body(s_col_ref, attr_ref, dist_ref, diff_ref,
                   W1s_ref, b1s_ref, W2s_ref, b2s_ref,
                   W1f_ref, b1f_ref, W2f_ref, b2f_ref,
                   gates_ref, msgs_ref, dir8_ref):
    dist = dist_ref[...]
    attr = attr_ref[...]
    freqs = jax.lax.broadcasted_iota(jnp.int32, (1, 20), 1).astype(jnp.float32) + 1.0
    radial = jnp.sin(dist * freqs * (jnp.pi / CUTOFF))
    cutoff_w = 0.5 * (jnp.cos(dist * (jnp.pi / CUTOFF)) + 1.0)

    h_f = _silu(jnp.dot(radial, W1f_ref[0:20, :], preferred_element_type=jnp.float32)
                + jnp.dot(attr, W1f_ref[20:36, :], preferred_element_type=jnp.float32)
                + b1f_ref[...])
    fw = (jnp.dot(h_f, W2f_ref[...], preferred_element_type=jnp.float32)
          + b2f_ref[...]) * cutoff_w

    h_s = _silu(jnp.dot(s_col_ref[...], W1s_ref[0:D, :], preferred_element_type=jnp.float32)
                + jnp.dot(attr, W1s_ref[D:D + DE, :], preferred_element_type=jnp.float32)
                + b1s_ref[...])
    s_j = jnp.dot(h_s, W2s_ref[...], preferred_element_type=jnp.float32) + b2s_ref[...]

    prod = s_j * fw
    gates_ref[...] = prod[:, 0:2 * D]
    msgs_ref[...] = prod[:, 2 * D:3 * D]
    dir3 = diff_ref[...] / (dist + 1e-07)
    dir8_ref[...] = jnp.concatenate(
        [dir3, jnp.zeros((BE, 125), jnp.float32)], axis=1)


def _tc_dense(s_col, edge_attr, edge_dist, edge_diff,
              W1s, b1s, W2s, b2s, W1f, b1f, W2f, b2f):
    nblk = E // BE
    full = lambda shape: pl.BlockSpec(shape, lambda i: (0, 0))
    grid_spec = pl.GridSpec(
        grid=(nblk,),
        in_specs=[
            pl.BlockSpec((BE, D), lambda i: (i, 0)),
            pl.BlockSpec((BE, DE), lambda i: (i, 0)),
            pl.BlockSpec((BE, 1), lambda i: (i, 0)),
            pl.BlockSpec((BE, 3), lambda i: (i, 0)),
            full((D + DE, D)), full((1, D)), full((D, 3 * D)), full((1, 3 * D)),
            full((36, D)), full((1, D)), full((D, 3 * D)), full((1, 3 * D)),
        ],
        out_specs=[
            pl.BlockSpec((BE, 2 * D), lambda i: (i, 0)),
            pl.BlockSpec((BE, D), lambda i: (i, 0)),
            pl.BlockSpec((BE, 128), lambda i: (i, 0)),
        ],
    )
    return pl.pallas_call(
        _tc_dense_body,
        grid_spec=grid_spec,
        out_shape=[
            jax.ShapeDtypeStruct((E, 2 * D), jnp.float32),
            jax.ShapeDtypeStruct((E, D), jnp.float32),
            jax.ShapeDtypeStruct((E, 128), jnp.float32),
        ],
    )(s_col, edge_attr, edge_dist.reshape(E, 1), edge_diff,
      W1s, b1s.reshape(1, D), W2s, b2s.reshape(1, 3 * D),
      W1f, b1f.reshape(1, D), W2f, b2f.reshape(1, 3 * D))


def kernel(s, vec, edge_index, edge_diff, edge_dist, edge_attr,
           W1s, b1s, W2s, b2s, W1f, b1f, W2f, b2f):
    row = edge_index[0]
    col = edge_index[1]

    s_col = _sc_gather_s(s, col)
    gates, msg_s, dir8 = _tc_dense(s_col, edge_attr, edge_dist, edge_diff,
                                   W1s, b1s, W2s, b2s, W1f, b1f, W2f, b2f)
    s_new = _sc_aggregate_s(msg_s, row, s)
    vec2d = vec.reshape(N, DV)
    vec_new = _sc_aggregate_v(gates, dir8, row, col, vec2d)
    return (s_new, vec_new.reshape(N, 3, D))
